# baseline (speedup 1.0000x reference)
import functools

import jax
import jax.numpy as jnp
from jax import lax
from jax.experimental import pallas as pl
from jax.experimental.pallas import tpu as pltpu
from jax.experimental.pallas import tpu_sc as plsc

D_MODEL = 64
NUM_WORKERS = 32
CHUNK = 128


def _build_kernel(batch: int):
    assert batch % NUM_WORKERS == 0
    b_per_w = batch // NUM_WORKERS
    assert b_per_w % CHUNK == 0
    n_chunks = b_per_w // CHUNK

    mesh = plsc.VectorSubcoreMesh(core_axis_name="c", subcore_axis_name="s")

    @functools.partial(
        pl.kernel,
        out_type=jax.ShapeDtypeStruct((batch, D_MODEL), jnp.float32),
        mesh=mesh,
        scratch_types=[
            pltpu.VMEM((b_per_w,), jnp.int32),
            [pltpu.VMEM((CHUNK, 128), jnp.float32) for _ in range(2)],
            [pltpu.VMEM((CHUNK, D_MODEL), jnp.float32) for _ in range(2)],
            [pltpu.VMEM((CHUNK,), jnp.int32) for _ in range(2)],
            [pltpu.SemaphoreType.DMA for _ in range(2)],
            [pltpu.SemaphoreType.DMA for _ in range(2)],
            [pltpu.SemaphoreType.DMA for _ in range(2)],
        ],
        compiler_params=pltpu.CompilerParams(needs_layout_passes=False),
    )
    def emb_kernel(table2_hbm, idxh_hbm, lo_hbm, out_hbm,
                   idxh_v, wide, outb, lo_v, gsem, wsem, lsem):
        wid = lax.axis_index("s") * 2 + lax.axis_index("c")
        base = wid * b_per_w
        pltpu.sync_copy(idxh_hbm.at[pl.ds(base, b_per_w)], idxh_v)

        def fire_gather(t, b):
            pltpu.async_copy(
                table2_hbm.at[idxh_v.at[pl.ds(t * CHUNK, CHUNK)]], wide[b],
                gsem[b])

        def drain_gather(t, b):
            pltpu.make_async_copy(
                table2_hbm.at[idxh_v.at[pl.ds(t * CHUNK, CHUNK)]], wide[b],
                gsem[b]).wait()

        def fire_lo(t, b):
            pltpu.async_copy(
                lo_hbm.at[pl.ds(base + t * CHUNK, CHUNK)], lo_v[b], lsem[b])

        def drain_lo(t, b):
            pltpu.make_async_copy(
                lo_hbm.at[pl.ds(base + t * CHUNK, CHUNK)], lo_v[b],
                lsem[b]).wait()

        def fire_write(t, b):
            pltpu.async_copy(
                outb[b], out_hbm.at[pl.ds(base + t * CHUNK, CHUNK)], wsem[b])

        def drain_write(t, b):
            pltpu.make_async_copy(
                outb[b], out_hbm.at[pl.ds(base + t * CHUNK, CHUNK)],
                wsem[b]).wait()

        fire_gather(0, 0)
        fire_lo(0, 0)

        @pl.loop(0, n_chunks, step=2)
        def _(t_base):
            for b in range(2):
                t = t_base + b
                drain_gather(t, b)
                drain_lo(t, b)

                @pl.when(t + 1 < n_chunks)
                def _():
                    fire_gather(t + 1, 1 - b)
                    fire_lo(t + 1, 1 - b)

                @pl.when(t >= 2)
                def _():
                    drain_write(t - 2, b)

                iota16 = lax.iota(jnp.int32, 16)
                zeros16 = jnp.zeros((16,), jnp.int32)

                @pl.loop(0, CHUNK, unroll=4)
                def _(c):
                    cv = zeros16 + c
                    offv = plsc.load_gather(lo_v[b], [cv])
                    for j0 in range(0, D_MODEL, 16):
                        colv = offv + (iota16 + j0)
                        outb[b][c, pl.ds(j0, 16)] = plsc.load_gather(
                            wide[b], [cv, colv])

                fire_write(t, b)

        drain_write(n_chunks - 2, 0)
        drain_write(n_chunks - 1, 1)

    return emb_kernel


def kernel(x, emb_table):
    b, s = x.shape
    flat_idx = x.reshape(b * s).astype(jnp.int32)
    idx_half = flat_idx >> 1
    lo = (flat_idx & 1) * D_MODEL
    table2 = emb_table.reshape(emb_table.shape[0] // 2, 2 * D_MODEL)
    out = _build_kernel(b * s)(table2, idx_half, lo)
    return out.reshape(b, s, D_MODEL)
